# baseline (speedup 1.0000x reference)
import jax
import jax.numpy as jnp
from jax import lax
from jax.experimental import pallas as pl
from jax.experimental.pallas import tpu as pltpu
from jax.experimental.pallas import tpu_sc as plsc

_N = 10000
_E = 320000
_D = 128

_NPAD = 10240
_ROWS_PER_TILE = _NPAD // 16
_NW = 32
_CHUNK = 128
_EPW = 10240
_NCHUNKS = _EPW // _CHUNK
_EPAD = _NW * _EPW

_NBUF = 2
_IDXBLK = 80
_NBLK = _NCHUNKS // _IDXBLK


def _seg_sum_body(h_hbm, src_hbm, dst_hbm, zero_hbm, out_hbm, ebuf_hbm,
                  idx, hsp, gsems, ssems, rows0, rows1):
    c = lax.axis_index("c")
    s = lax.axis_index("s")
    wid = s * 2 + c
    rows = (rows0, rows1)
    tile_rows = pl.ds(s * _ROWS_PER_TILE, _ROWS_PER_TILE)
    ngroups = _IDXBLK // _NBUF

    pltpu.sync_copy(h_hbm.at[tile_rows], hsp.at[tile_rows])
    plsc.subcore_barrier()

    for blk in range(_NBLK):
        cbase = wid * _NCHUNKS + blk * _IDXBLK
        pltpu.sync_copy(src_hbm.at[pl.ds(cbase, _IDXBLK)], idx)
        for b in range(_NBUF):
            pltpu.async_copy(hsp.at[idx.at[b]], rows[b], gsems[b])

        def g_group(k, carry, cbase=cbase):
            for b in range(_NBUF):
                g = k * _NBUF + b
                pltpu.make_async_copy(hsp.at[idx.at[g]], rows[b],
                                      gsems[b]).wait()
                ebuf_rows = ebuf_hbm.at[pl.ds((cbase + g) * _CHUNK, _CHUNK)]
                pltpu.async_copy(rows[b], ebuf_rows, ssems[b])

                @pl.when(k < ngroups - 1)
                def _():
                    pltpu.make_async_copy(rows[b], ebuf_rows, ssems[b]).wait()
                    pltpu.async_copy(hsp.at[idx.at[g + _NBUF]], rows[b],
                                     gsems[b])

                @pl.when(k == ngroups - 1)
                def _():
                    pltpu.make_async_copy(rows[b], ebuf_rows, ssems[b]).wait()
            return carry

        lax.fori_loop(0, ngroups, g_group, 0)

    plsc.subcore_barrier()

    pltpu.sync_copy(zero_hbm, hsp.at[tile_rows])
    plsc.subcore_barrier()

    for blk in range(_NBLK):
        cbase = wid * _NCHUNKS + blk * _IDXBLK
        pltpu.sync_copy(dst_hbm.at[pl.ds(cbase, _IDXBLK)], idx)
        for b in range(_NBUF):
            pltpu.async_copy(ebuf_hbm.at[pl.ds((cbase + b) * _CHUNK, _CHUNK)],
                             rows[b], gsems[b])

        def s_group(k, carry, cbase=cbase):
            for b in range(_NBUF):
                g = k * _NBUF + b
                pltpu.make_async_copy(
                    ebuf_hbm.at[pl.ds((cbase + g) * _CHUNK, _CHUNK)],
                    rows[b], gsems[b]).wait()
                pltpu.async_copy(rows[b], hsp.at[idx.at[g]], ssems[b],
                                 add=True)
            for b in range(_NBUF):
                g = k * _NBUF + b
                pltpu.make_async_copy(rows[b], hsp.at[idx.at[g]],
                                      ssems[b]).wait()

                @pl.when(k < ngroups - 1)
                def _():
                    pltpu.async_copy(
                        ebuf_hbm.at[pl.ds((cbase + g + _NBUF) * _CHUNK,
                                          _CHUNK)],
                        rows[b], gsems[b])
            return carry

        lax.fori_loop(0, ngroups, s_group, 0)

    plsc.subcore_barrier()
    pltpu.sync_copy(hsp.at[tile_rows], out_hbm.at[c, tile_rows])


_seg_sum = pl.kernel(
    _seg_sum_body,
    out_type=[
        jax.ShapeDtypeStruct((2, _NPAD, _D), jnp.float32),
        jax.ShapeDtypeStruct((_EPAD, _D), jnp.float32),
    ],
    mesh=plsc.VectorSubcoreMesh(core_axis_name="c", subcore_axis_name="s"),
    scratch_types=[
        pltpu.VMEM((_IDXBLK, _CHUNK), jnp.int32),
        pltpu.VMEM_SHARED((_NPAD, _D), jnp.float32),
        [pltpu.SemaphoreType.DMA] * _NBUF,
        [pltpu.SemaphoreType.DMA] * _NBUF,
        pltpu.VMEM((_CHUNK, _D), jnp.float32),
        pltpu.VMEM((_CHUNK, _D), jnp.float32),
    ],
)


def _mlp_body(h_ref, p_ref, w1_ref, b1_ref, w2_ref, b2_ref, o_ref):
    z = h_ref[...] + p_ref[0] + p_ref[1]
    a = jnp.dot(z, w1_ref[...], preferred_element_type=jnp.float32) + b1_ref[...]
    a = jnp.maximum(a, 0.0)
    o_ref[...] = jnp.dot(a, w2_ref[...], preferred_element_type=jnp.float32) + b2_ref[...]


_BLK = 1280


def _mlp(h, p, w1, b1, w2, b2):
    grid = (_NPAD // _BLK,)
    row_spec = pl.BlockSpec((_BLK, _D), lambda i: (i, 0))
    p_spec = pl.BlockSpec((2, _BLK, _D), lambda i: (0, i, 0))
    full = pl.BlockSpec((_D, _D), lambda i: (0, 0))
    bias = pl.BlockSpec((1, _D), lambda i: (0, 0))
    return pl.pallas_call(
        _mlp_body,
        grid=grid,
        in_specs=[row_spec, p_spec, full, bias, full, bias],
        out_specs=row_spec,
        out_shape=jax.ShapeDtypeStruct((_NPAD, _D), jnp.float32),
    )(h, p, w1, b1, w2, b2)


def kernel(x, edge_index, W1_0, b1_0, W2_0, b2_0, W1_1, b1_1, W2_1, b2_1,
           W1_2, b1_2, W2_2, b2_2):
    src = edge_index[0]
    dst = edge_index[1]
    pad = _EPAD - _E
    src_p = jnp.concatenate([src, jnp.zeros((pad,), jnp.int32)])
    src_p = src_p.reshape(_NW * _NCHUNKS, _CHUNK)
    pad_dst = _N + jnp.arange(pad, dtype=jnp.int32) % (_NPAD - _N)
    dst_p = jnp.concatenate([dst, pad_dst])
    dst_p = dst_p.reshape(_NW * _NCHUNKS, _CHUNK)
    h = jnp.pad(x, ((0, _NPAD - _N), (0, 0)))
    zeros = jnp.zeros((_ROWS_PER_TILE, _D), jnp.float32)

    params = [(W1_0, b1_0, W2_0, b2_0), (W1_1, b1_1, W2_1, b2_1),
              (W1_2, b1_2, W2_2, b2_2)]
    for (w1, b1, w2, b2) in params:
        parts, _ = _seg_sum(h, src_p, dst_p, zeros)
        h = _mlp(h, parts, w1, b1.reshape(1, _D), w2, b2.reshape(1, _D))
    return h[:_N]


# speedup vs baseline: 1.2071x; 1.2071x over previous
import jax
import jax.numpy as jnp
from jax import lax
from jax.experimental import pallas as pl
from jax.experimental.pallas import tpu as pltpu
from jax.experimental.pallas import tpu_sc as plsc

_N = 10000
_E = 320000
_D = 128

_NPAD = 10240
_ROWS_PER_TILE = _NPAD // 16
_NW = 32
_CHUNK = 128
_EPW = 10240
_NCHUNKS = _EPW // _CHUNK
_EPAD = _NW * _EPW

_NBUF = 2
_IDXBLK = 80
_NBLK = _NCHUNKS // _IDXBLK


def _seg_sum_body(h_hbm, src_hbm, dst_hbm, zero_hbm, out_hbm, ebuf_hbm,
                  idx, hsp, gsems, ssems, rows0, rows1):
    c = lax.axis_index("c")
    s = lax.axis_index("s")
    wid = s * 2 + c
    rows = (rows0, rows1)
    tile_rows = pl.ds(s * _ROWS_PER_TILE, _ROWS_PER_TILE)
    ngroups = _IDXBLK // _NBUF

    pltpu.sync_copy(h_hbm.at[tile_rows], hsp.at[tile_rows])
    plsc.subcore_barrier()

    for blk in range(_NBLK):
        cbase = wid * _NCHUNKS + blk * _IDXBLK
        pltpu.sync_copy(src_hbm.at[pl.ds(cbase, _IDXBLK)], idx)
        for b in range(_NBUF):
            pltpu.async_copy(hsp.at[idx.at[b]], rows[b], gsems[b])

        def g_group(k, carry, cbase=cbase):
            for b in range(_NBUF):
                g = k * _NBUF + b
                pltpu.make_async_copy(hsp.at[idx.at[g]], rows[b],
                                      gsems[b]).wait()
                ebuf_rows = ebuf_hbm.at[pl.ds((cbase + g) * _CHUNK, _CHUNK)]
                pltpu.async_copy(rows[b], ebuf_rows, ssems[b])

                @pl.when(k < ngroups - 1)
                def _():
                    pltpu.make_async_copy(rows[b], ebuf_rows, ssems[b]).wait()
                    pltpu.async_copy(hsp.at[idx.at[g + _NBUF]], rows[b],
                                     gsems[b])

                @pl.when(k == ngroups - 1)
                def _():
                    pltpu.make_async_copy(rows[b], ebuf_rows, ssems[b]).wait()
            return carry

        lax.fori_loop(0, ngroups, g_group, 0)

    plsc.subcore_barrier()

    pltpu.sync_copy(zero_hbm, hsp.at[tile_rows])
    plsc.subcore_barrier()

    for blk in range(_NBLK):
        cbase = wid * _NCHUNKS + blk * _IDXBLK
        pltpu.sync_copy(dst_hbm.at[pl.ds(cbase, _IDXBLK)], idx)
        for b in range(_NBUF):
            pltpu.async_copy(ebuf_hbm.at[pl.ds((cbase + b) * _CHUNK, _CHUNK)],
                             rows[b], gsems[b])

        def s_group(k, carry, cbase=cbase):
            for b in range(_NBUF):
                g = k * _NBUF + b
                pltpu.make_async_copy(
                    ebuf_hbm.at[pl.ds((cbase + g) * _CHUNK, _CHUNK)],
                    rows[b], gsems[b]).wait()
                pltpu.sync_copy(rows[b], hsp.at[idx.at[g]], add=True)

                @pl.when(k < ngroups - 1)
                def _():
                    pltpu.async_copy(
                        ebuf_hbm.at[pl.ds((cbase + g + _NBUF) * _CHUNK,
                                          _CHUNK)],
                        rows[b], gsems[b])
            return carry

        lax.fori_loop(0, ngroups, s_group, 0)

    plsc.subcore_barrier()
    pltpu.sync_copy(hsp.at[tile_rows], out_hbm.at[c, tile_rows])


_seg_sum = pl.kernel(
    _seg_sum_body,
    out_type=[
        jax.ShapeDtypeStruct((2, _NPAD, _D), jnp.float32),
        jax.ShapeDtypeStruct((_EPAD, _D), jnp.float32),
    ],
    mesh=plsc.VectorSubcoreMesh(core_axis_name="c", subcore_axis_name="s"),
    scratch_types=[
        pltpu.VMEM((_IDXBLK, _CHUNK), jnp.int32),
        pltpu.VMEM_SHARED((_NPAD, _D), jnp.float32),
        [pltpu.SemaphoreType.DMA] * _NBUF,
        [pltpu.SemaphoreType.DMA] * _NBUF,
        pltpu.VMEM((_CHUNK, _D), jnp.float32),
        pltpu.VMEM((_CHUNK, _D), jnp.float32),
    ],
)


def _mlp_body(h_ref, p_ref, w1_ref, b1_ref, w2_ref, b2_ref, o_ref):
    z = h_ref[...] + p_ref[0] + p_ref[1]
    a = jnp.dot(z, w1_ref[...], preferred_element_type=jnp.float32) + b1_ref[...]
    a = jnp.maximum(a, 0.0)
    o_ref[...] = jnp.dot(a, w2_ref[...], preferred_element_type=jnp.float32) + b2_ref[...]


_BLK = 1280


def _mlp(h, p, w1, b1, w2, b2):
    grid = (_NPAD // _BLK,)
    row_spec = pl.BlockSpec((_BLK, _D), lambda i: (i, 0))
    p_spec = pl.BlockSpec((2, _BLK, _D), lambda i: (0, i, 0))
    full = pl.BlockSpec((_D, _D), lambda i: (0, 0))
    bias = pl.BlockSpec((1, _D), lambda i: (0, 0))
    return pl.pallas_call(
        _mlp_body,
        grid=grid,
        in_specs=[row_spec, p_spec, full, bias, full, bias],
        out_specs=row_spec,
        out_shape=jax.ShapeDtypeStruct((_NPAD, _D), jnp.float32),
    )(h, p, w1, b1, w2, b2)


def kernel(x, edge_index, W1_0, b1_0, W2_0, b2_0, W1_1, b1_1, W2_1, b2_1,
           W1_2, b1_2, W2_2, b2_2):
    src = edge_index[0]
    dst = edge_index[1]
    pad = _EPAD - _E
    src_p = jnp.concatenate([src, jnp.zeros((pad,), jnp.int32)])
    src_p = src_p.reshape(_NW * _NCHUNKS, _CHUNK)
    pad_dst = _N + jnp.arange(pad, dtype=jnp.int32) % (_NPAD - _N)
    dst_p = jnp.concatenate([dst, pad_dst])
    dst_p = dst_p.reshape(_NW * _NCHUNKS, _CHUNK)
    h = jnp.pad(x, ((0, _NPAD - _N), (0, 0)))
    zeros = jnp.zeros((_ROWS_PER_TILE, _D), jnp.float32)

    params = [(W1_0, b1_0, W2_0, b2_0), (W1_1, b1_1, W2_1, b2_1),
              (W1_2, b1_2, W2_2, b2_2)]
    for (w1, b1, w2, b2) in params:
        parts, _ = _seg_sum(h, src_p, dst_p, zeros)
        h = _mlp(h, parts, w1, b1.reshape(1, _D), w2, b2.reshape(1, _D))
    return h[:_N]
